# baseline (speedup 1.0000x reference)
import functools
import numpy as np
import jax
import jax.numpy as jnp
from jax.experimental import pallas as pl
from jax.experimental.pallas import tpu as pltpu

B, T, D = 4, 8192, 768
K = 32
N1, N2 = 64, 128
DB = 128

_HIGHEST = jax.lax.Precision.HIGHEST


def _mm(a, b):
    return jax.lax.dot(a, b, precision=_HIGHEST,
                       preferred_element_type=jnp.float32)


def _build_consts():
    t1 = np.arange(N1)
    t2 = np.arange(N2)
    f2 = np.arange(N2)
    f1 = np.arange(N1 // 2)
    w2c = np.cos(2 * np.pi * np.outer(f2, t2) / N2)
    w2s = -np.sin(2 * np.pi * np.outer(f2, t2) / N2)
    twc = np.cos(2 * np.pi * np.outer(f2, t1) / T)
    tws = -np.sin(2 * np.pi * np.outer(f2, t1) / T)
    w1c = np.cos(2 * np.pi * np.outer(f1, t1) / N1)
    w1s = -np.sin(2 * np.pi * np.outer(f1, t1) / N1)
    n = np.arange(T)
    r = np.arange(K)
    sync = np.cos(2 * np.pi * np.outer(n, r) / T) / T
    sync[:, 1:] *= 2.0
    syns = -2.0 * np.sin(2 * np.pi * np.outer(n, r) / T) / T
    syns[:, 0] = 0.0
    f32 = lambda a: jnp.asarray(a, dtype=jnp.float32)
    return (f32(w2c), f32(w2s), f32(twc), f32(tws), f32(w1c), f32(w1s),
            f32(sync), f32(syns))


def _fourier_kernel(x_ref, w2c_ref, w2s_ref, twc_ref, tws_ref,
                    w1c_ref, w1s_ref, sync_ref, syns_ref, o_ref):
    x = x_ref[0]
    mean = jnp.mean(x, axis=0, keepdims=True)
    xc = x - mean
    var = jnp.sum(xc * xc, axis=0, keepdims=True) / (T - 1)
    std = jnp.sqrt(var) + 1e-8
    xs = jnp.clip(xc / std, -2.0, 2.0) + 1e-9

    a = xs.reshape(N2, N1 * DB)
    yre = _mm(w2c_ref[...], a)
    yim = _mm(w2s_ref[...], a)
    y3re = yre.reshape(N2, N1, DB)
    y3im = yim.reshape(N2, N1, DB)
    twc = twc_ref[...][:, :, None]
    tws = tws_ref[...][:, :, None]
    zre = y3re * twc - y3im * tws
    zim = y3re * tws + y3im * twc
    zre_t = jnp.transpose(zre, (1, 0, 2)).reshape(N1, N2 * DB)
    zim_t = jnp.transpose(zim, (1, 0, 2)).reshape(N1, N2 * DB)
    w1c = w1c_ref[...]
    w1s = w1s_ref[...]
    xre = (_mm(w1c, zre_t) - _mm(w1s, zim_t)).reshape(T // 2, DB)
    xim = (_mm(w1c, zim_t) + _mm(w1s, zre_t)).reshape(T // 2, DB)

    amp2 = xre * xre + xim * xim
    row = jax.lax.broadcasted_iota(jnp.int32, (T // 2, DB), 0)
    amp2 = jnp.where(row == 0, -1.0, amp2)
    G, NC = 32, 12
    amp3 = amp2.reshape(G, (T // 2) // G, DB)
    row3 = row.reshape(G, (T // 2) // G, DB)
    xre3 = xre.reshape(G, (T // 2) // G, DB)
    xim3 = xim.reshape(G, (T // 2) // G, DB)
    cv, ci, cr, cm = [], [], [], []
    for _ in range(NC):
        gm = jnp.max(amp3, axis=1)
        gi = jnp.min(jnp.where(amp3 == gm[:, None, :], row3, T),
                     axis=1)
        hit3 = row3 == gi[:, None, :]
        cv.append(gm)
        ci.append(gi)
        cr.append(jnp.sum(jnp.where(hit3, xre3, 0.0), axis=1))
        cm.append(jnp.sum(jnp.where(hit3, xim3, 0.0), axis=1))
        amp3 = jnp.where(hit3, -1.0, amp3)
    cval = jnp.stack(cv, axis=0).reshape(NC * G, DB)
    cidx = jnp.stack(ci, axis=0).reshape(NC * G, DB)
    cre = jnp.stack(cr, axis=0).reshape(NC * G, DB)
    cim = jnp.stack(cm, axis=0).reshape(NC * G, DB)
    sel_re = []
    sel_im = []
    for _ in range(K):
        m = jnp.max(cval, axis=0, keepdims=True)
        idx = jnp.min(jnp.where(cval == m, cidx, T), axis=0,
                      keepdims=True)
        hit = cidx == idx
        sel_re.append(jnp.sum(jnp.where(hit, cre, 0.0), axis=0))
        sel_im.append(jnp.sum(jnp.where(hit, cim, 0.0), axis=0))
        cval = jnp.where(hit, -1.0, cval)
    are = jnp.stack(sel_re, axis=0)
    aim = jnp.stack(sel_im, axis=0)

    out = _mm(sync_ref[...], are) + _mm(syns_ref[...], aim)
    o_ref[0] = out * std + mean


@jax.jit
def kernel(x):
    consts = _build_consts()
    grid = (B, D // DB)
    in_specs = [pl.BlockSpec((1, T, DB), lambda b, j: (b, 0, j))]
    for c in consts:
        in_specs.append(
            pl.BlockSpec(c.shape, functools.partial(
                lambda nd, b, j: (0,) * nd, len(c.shape))))
    return pl.pallas_call(
        _fourier_kernel,
        grid=grid,
        in_specs=in_specs,
        out_specs=pl.BlockSpec((1, T, DB), lambda b, j: (b, 0, j)),
        out_shape=jax.ShapeDtypeStruct((B, T, D), jnp.float32),
        compiler_params=pltpu.CompilerParams(
            dimension_semantics=("parallel", "parallel")),
    )(x, *consts)


# speedup vs baseline: 1.1236x; 1.1236x over previous
import functools
import numpy as np
import jax
import jax.numpy as jnp
from jax.experimental import pallas as pl
from jax.experimental.pallas import tpu as pltpu

B, T, D = 4, 8192, 768
K = 32
N1, N2 = 64, 128
DB = 128

_HIGHEST = jax.lax.Precision.HIGHEST


def _mm(a, b):
    return jax.lax.dot(a, b, precision=_HIGHEST,
                       preferred_element_type=jnp.float32)


def _build_consts():
    t1 = np.arange(N1)
    t2 = np.arange(N2)
    f2 = np.arange(N2)
    f1 = np.arange(N1 // 2)
    w2c = np.cos(2 * np.pi * np.outer(f2, t2) / N2)
    w2s = -np.sin(2 * np.pi * np.outer(f2, t2) / N2)
    twc = np.cos(2 * np.pi * np.outer(f2, t1) / T)
    tws = -np.sin(2 * np.pi * np.outer(f2, t1) / T)
    w1c = np.cos(2 * np.pi * np.outer(f1, t1) / N1)
    w1s = -np.sin(2 * np.pi * np.outer(f1, t1) / N1)
    n = np.arange(T)
    r = np.arange(K)
    sync = np.cos(2 * np.pi * np.outer(n, r) / T) / T
    sync[:, 1:] *= 2.0
    syns = -2.0 * np.sin(2 * np.pi * np.outer(n, r) / T) / T
    syns[:, 0] = 0.0
    f32 = lambda a: jnp.asarray(a, dtype=jnp.float32)
    return (f32(w2c), f32(w2s), f32(twc), f32(tws), f32(w1c), f32(w1s),
            f32(sync), f32(syns))


def _fourier_kernel(x_ref, w2c_ref, w2s_ref, twc_ref, tws_ref,
                    w1c_ref, w1s_ref, sync_ref, syns_ref, o_ref):
    x = x_ref[0]
    mean = jnp.mean(x, axis=0, keepdims=True)
    xc = x - mean
    var = jnp.sum(xc * xc, axis=0, keepdims=True) / (T - 1)
    std = jnp.sqrt(var) + 1e-8
    xs = jnp.clip(xc / std, -2.0, 2.0) + 1e-9

    a = xs.reshape(N2, N1 * DB)
    yre = _mm(w2c_ref[...], a)
    yim = _mm(w2s_ref[...], a)
    y3re = yre.reshape(N2, N1, DB)
    y3im = yim.reshape(N2, N1, DB)
    twc = twc_ref[...][:, :, None]
    tws = tws_ref[...][:, :, None]
    zre = y3re * twc - y3im * tws
    zim = y3re * tws + y3im * twc
    zre_t = jnp.transpose(zre, (1, 0, 2)).reshape(N1, N2 * DB)
    zim_t = jnp.transpose(zim, (1, 0, 2)).reshape(N1, N2 * DB)
    w1c = w1c_ref[...]
    w1s = w1s_ref[...]
    xre = (_mm(w1c, zre_t) - _mm(w1s, zim_t)).reshape(T // 2, DB)
    xim = (_mm(w1c, zim_t) + _mm(w1s, zre_t)).reshape(T // 2, DB)

    amp2 = xre * xre + xim * xim
    row = jax.lax.broadcasted_iota(jnp.int32, (T // 2, DB), 0)
    amp2 = jnp.where(row == 0, -1.0, amp2)
    sel_re = []
    sel_im = []
    for _ in range(K):
        m = jnp.max(amp2, axis=0, keepdims=True)
        idx = jnp.min(jnp.where(amp2 == m, row, T), axis=0,
                      keepdims=True)
        hit = row == idx
        sel_re.append(jnp.sum(jnp.where(hit, xre, 0.0), axis=0))
        sel_im.append(jnp.sum(jnp.where(hit, xim, 0.0), axis=0))
        amp2 = jnp.where(hit, -1.0, amp2)
    are = jnp.stack(sel_re, axis=0)
    aim = jnp.stack(sel_im, axis=0)

    out = _mm(sync_ref[...], are) + _mm(syns_ref[...], aim)
    o_ref[0] = out * std + mean


@jax.jit
def kernel(x):
    consts = _build_consts()
    grid = (B, D // DB)
    in_specs = [pl.BlockSpec((1, T, DB), lambda b, j: (b, 0, j))]
    for c in consts:
        in_specs.append(
            pl.BlockSpec(c.shape, functools.partial(
                lambda nd, b, j: (0,) * nd, len(c.shape))))
    return pl.pallas_call(
        _fourier_kernel,
        grid=grid,
        in_specs=in_specs,
        out_specs=pl.BlockSpec((1, T, DB), lambda b, j: (b, 0, j)),
        out_shape=jax.ShapeDtypeStruct((B, T, D), jnp.float32),
        compiler_params=pltpu.CompilerParams(
            dimension_semantics=("parallel", "parallel")),
    )(x, *consts)


# speedup vs baseline: 1.2464x; 1.1093x over previous
import functools
import numpy as np
import jax
import jax.numpy as jnp
from jax.experimental import pallas as pl
from jax.experimental.pallas import tpu as pltpu

B, T, D = 4, 8192, 768
K = 32
N1, N2 = 64, 128
DB = 128

_HIGHEST = jax.lax.Precision.HIGHEST


def _mm(a, b):
    return jax.lax.dot(a, b, precision=_HIGHEST,
                       preferred_element_type=jnp.float32)


def _build_consts():
    t1 = np.arange(N1)
    t2 = np.arange(N2)
    f2 = np.arange(N2)
    f1 = np.arange(N1 // 2)
    w2c = np.cos(2 * np.pi * np.outer(f2, t2) / N2)
    w2s = -np.sin(2 * np.pi * np.outer(f2, t2) / N2)
    twc = np.cos(2 * np.pi * np.outer(f2, t1) / T)
    tws = -np.sin(2 * np.pi * np.outer(f2, t1) / T)
    w1c = np.cos(2 * np.pi * np.outer(f1, t1) / N1)
    w1s = -np.sin(2 * np.pi * np.outer(f1, t1) / N1)
    n = np.arange(T)
    r = np.arange(K)
    sync = np.cos(2 * np.pi * np.outer(n, r) / T) / T
    sync[:, 1:] *= 2.0
    syns = -2.0 * np.sin(2 * np.pi * np.outer(n, r) / T) / T
    syns[:, 0] = 0.0
    f32 = lambda a: jnp.asarray(a, dtype=jnp.float32)
    return (f32(w2c), f32(w2s), f32(twc), f32(tws), f32(w1c), f32(w1s),
            f32(sync), f32(syns))


def _fourier_kernel(x_ref, w2c_ref, w2s_ref, twc_ref, tws_ref,
                    w1c_ref, w1s_ref, sync_ref, syns_ref, o_ref):
    x = x_ref[0]
    mean = jnp.mean(x, axis=0, keepdims=True)
    xc = x - mean
    var = jnp.sum(xc * xc, axis=0, keepdims=True) / (T - 1)
    std = jnp.sqrt(var) + 1e-8
    xs = jnp.clip(xc / std, -2.0, 2.0) + 1e-9

    a = xs.reshape(N2, N1 * DB)
    yre = _mm(w2c_ref[...], a)
    yim = _mm(w2s_ref[...], a)
    y3re = yre.reshape(N2, N1, DB)
    y3im = yim.reshape(N2, N1, DB)
    twc = twc_ref[...][:, :, None]
    tws = tws_ref[...][:, :, None]
    zre = y3re * twc - y3im * tws
    zim = y3re * tws + y3im * twc
    zre_t = jnp.transpose(zre, (1, 0, 2)).reshape(N1, N2 * DB)
    zim_t = jnp.transpose(zim, (1, 0, 2)).reshape(N1, N2 * DB)
    w1c = w1c_ref[...]
    w1s = w1s_ref[...]
    xre = (_mm(w1c, zre_t) - _mm(w1s, zim_t)).reshape(T // 2, DB)
    xim = (_mm(w1c, zim_t) + _mm(w1s, zre_t)).reshape(T // 2, DB)

    amp2 = xre * xre + xim * xim
    row0 = jax.lax.broadcasted_iota(jnp.int32, (T // 2, DB), 0)
    amp2 = jnp.where(row0 == 0, -1.0, amp2)
    H = T // 4
    a_lo, a_hi = amp2[:H], amp2[H:]
    re_lo, re_hi = xre[:H], xre[H:]
    im_lo, im_hi = xim[:H], xim[H:]
    prow = jax.lax.broadcasted_iota(jnp.int32, (H, DB), 0)
    which = a_lo >= a_hi
    wval = jnp.where(which, a_lo, a_hi)
    wre = jnp.where(which, re_lo, re_hi)
    wim = jnp.where(which, im_lo, im_hi)
    wrow = prow + jnp.where(which, 0, H)
    lval = jnp.where(which, a_hi, a_lo)
    lre = jnp.where(which, re_hi, re_lo)
    lim_ = jnp.where(which, im_hi, im_lo)
    lrow = prow + jnp.where(which, H, 0)
    sel_re = []
    sel_im = []
    for _ in range(K):
        m = jnp.max(wval, axis=0, keepdims=True)
        idx = jnp.min(jnp.where(wval == m, wrow, T), axis=0,
                      keepdims=True)
        hit = wrow == idx
        sel_re.append(jnp.sum(jnp.where(hit, wre, 0.0), axis=0))
        sel_im.append(jnp.sum(jnp.where(hit, wim, 0.0), axis=0))
        wval = jnp.where(hit, lval, wval)
        wre = jnp.where(hit, lre, wre)
        wim = jnp.where(hit, lim_, wim)
        wrow = jnp.where(hit, lrow, wrow)
        lval = jnp.where(hit, -1.0, lval)
    are = jnp.stack(sel_re, axis=0)
    aim = jnp.stack(sel_im, axis=0)

    out = _mm(sync_ref[...], are) + _mm(syns_ref[...], aim)
    o_ref[0] = out * std + mean


@jax.jit
def kernel(x):
    consts = _build_consts()
    grid = (B, D // DB)
    in_specs = [pl.BlockSpec((1, T, DB), lambda b, j: (b, 0, j))]
    for c in consts:
        in_specs.append(
            pl.BlockSpec(c.shape, functools.partial(
                lambda nd, b, j: (0,) * nd, len(c.shape))))
    return pl.pallas_call(
        _fourier_kernel,
        grid=grid,
        in_specs=in_specs,
        out_specs=pl.BlockSpec((1, T, DB), lambda b, j: (b, 0, j)),
        out_shape=jax.ShapeDtypeStruct((B, T, D), jnp.float32),
        compiler_params=pltpu.CompilerParams(
            dimension_semantics=("parallel", "parallel")),
    )(x, *consts)
